# baseline (speedup 1.0000x reference)
import functools

import jax
import jax.numpy as jnp
from jax import lax
from jax.experimental import pallas as pl
from jax.experimental.pallas import tpu as pltpu
from jax.experimental.pallas import tpu_sc as plsc

SCALE_BOUND = 0.11
LIKELIHOOD_BOUND = 1e-09

_MAGIC = 12582912.0
_BIG = 8388608.0

_P = 0.3275911
_A1 = 0.254829592
_A2 = -0.284496736
_A3 = 1.421413741
_A4 = -1.453152027
_A5 = 1.061405429
_INV_SQRT2 = 0.7071067811865476

_B = 8
_R = 1536
_NW = 32
_RW = (_B * _R) // _NW
_CH = 128
_NCH = _RW // _CH


def _round_nte(d):
    q = (d + _MAGIC) - _MAGIC
    return jnp.where(jnp.abs(d) >= _BIG, d, q)


def _sc_body(x_hbm, m_hbm, out_hbm, xb, mb, ob, si0, si1, so0, so1):
    wid = lax.axis_index("s") * 2 + lax.axis_index("c")
    row0 = wid * _RW
    b = row0 // _R
    r_in_b = row0 - b * _R
    sem_in = (si0, si1)
    sem_out = (so0, so1)

    def start_in(k):
        buf = k % 2
        r0 = r_in_b + k * _CH
        return [
            pltpu.async_copy(x_hbm.at[b, pl.ds(r0, _CH)], xb.at[buf], sem_in[buf]),
            pltpu.async_copy(m_hbm.at[b, pl.ds(r0, _CH)], mb.at[buf], sem_in[buf]),
        ]

    hin = [None] * _NCH
    hout = [None] * _NCH
    hin[0] = start_in(0)
    for k in range(_NCH):
        cur = k % 2
        if k + 1 < _NCH:
            hin[k + 1] = start_in(k + 1)
        for h in hin[k]:
            h.wait()
        if k >= 2:
            for h in hout[k - 2]:
                h.wait()

        @plsc.parallel_loop(0, _CH, 1)
        def row(r):
            for j in range(8):
                sl = pl.ds(j * 16, 16)
                mv = mb[cur, r, sl]
                ob[cur, r, sl] = _round_nte(xb[cur, r, sl] - mv) + mv

        r0 = r_in_b + k * _CH
        hout[k] = [
            pltpu.async_copy(ob.at[cur], out_hbm.at[b, pl.ds(r0, _CH)], sem_out[cur]),
        ]
    for h in hout[_NCH - 2]:
        h.wait()
    for h in hout[_NCH - 1]:
        h.wait()


def _erfc_nonneg(a):
    t = 1.0 / (1.0 + _P * a)
    poly = t * (_A1 + t * (_A2 + t * (_A3 + t * (_A4 + t * _A5))))
    return poly * jnp.exp(-(a * a))


def _tc_body(x_ref, s_ref, m_ref, lik_ref):
    x = x_ref[...]
    s = s_ref[...]
    m = m_ref[...]
    v = jnp.abs(_round_nte(x - m))
    sb = jnp.maximum(s, SCALE_BOUND)
    inv = _INV_SQRT2 / sb
    a = (v + 0.5) * inv
    bz = (v - 0.5) * inv
    ea = _erfc_nonneg(a)
    eb_mag = _erfc_nonneg(jnp.abs(bz))
    eb = jnp.where(bz < 0.0, 2.0 - eb_mag, eb_mag)
    lik = 0.5 * (eb - ea)
    lik_ref[...] = jnp.maximum(lik, LIKELIHOOD_BOUND)


def kernel(x, scales, means):
    shape = x.shape
    r3 = (_B, _R, 128)
    x3 = x.reshape(r3)
    s3 = scales.reshape(r3)
    m3 = means.reshape(r3)

    bb, br = 2, 768
    spec = pl.BlockSpec((bb, br, 128), lambda i, j: (i, j, 0))
    lik = pl.pallas_call(
        _tc_body,
        grid=(_B // bb, _R // br),
        in_specs=[spec, spec, spec],
        out_specs=spec,
        out_shape=jax.ShapeDtypeStruct(r3, jnp.float32),
    )(x3, s3, m3)

    mesh = plsc.VectorSubcoreMesh(core_axis_name="c", subcore_axis_name="s")
    out = functools.partial(
        pl.kernel,
        mesh=mesh,
        out_type=jax.ShapeDtypeStruct(r3, jnp.float32),
        scratch_types=[
            pltpu.VMEM((2, _CH, 128), jnp.float32),
            pltpu.VMEM((2, _CH, 128), jnp.float32),
            pltpu.VMEM((2, _CH, 128), jnp.float32),
            pltpu.SemaphoreType.DMA,
            pltpu.SemaphoreType.DMA,
            pltpu.SemaphoreType.DMA,
            pltpu.SemaphoreType.DMA,
        ],
    )(_sc_body)(x3, m3)

    return out.reshape(shape), lik.reshape(shape)


# speedup vs baseline: 1.0041x; 1.0041x over previous
import functools

import jax
import jax.numpy as jnp
from jax import lax
from jax.experimental import pallas as pl
from jax.experimental.pallas import tpu as pltpu
from jax.experimental.pallas import tpu_sc as plsc

SCALE_BOUND = 0.11
LIKELIHOOD_BOUND = 1e-09

_MAGIC = 12582912.0
_BIG = 8388608.0

_P = 0.3275911
_A1 = 0.254829592
_A2 = -0.284496736
_A3 = 1.421413741
_A4 = -1.453152027
_A5 = 1.061405429
_INV_SQRT2 = 0.7071067811865476

_B = 8
_R = 1536
_NW = 32
_RW = (_B * _R) // _NW
_CH = 128
_NCH = _RW // _CH


def _round_nte(d):
    q = (d + _MAGIC) - _MAGIC
    return jnp.where(jnp.abs(d) >= _BIG, d, q)


def _sc_body(x_hbm, m_hbm, out_hbm, xb, mb, ob, si0, si1, so0, so1):
    wid = lax.axis_index("s") * 2 + lax.axis_index("c")
    row0 = wid * _RW
    b = row0 // _R
    r_in_b = row0 - b * _R
    sem_in = (si0, si1)
    sem_out = (so0, so1)

    def start_in(k):
        buf = k % 2
        r0 = r_in_b + k * _CH
        return [
            pltpu.async_copy(x_hbm.at[b, pl.ds(r0, _CH)], xb.at[buf], sem_in[buf]),
            pltpu.async_copy(m_hbm.at[b, pl.ds(r0, _CH)], mb.at[buf], sem_in[buf]),
        ]

    hin = [None] * _NCH
    hout = [None] * _NCH
    hin[0] = start_in(0)
    for k in range(_NCH):
        cur = k % 2
        if k + 1 < _NCH:
            hin[k + 1] = start_in(k + 1)
        for h in hin[k]:
            h.wait()
        if k >= 2:
            for h in hout[k - 2]:
                h.wait()

        @plsc.parallel_loop(0, _CH, 1)
        def row(r):
            for j in range(8):
                sl = pl.ds(j * 16, 16)
                mv = mb[cur, r, sl]
                ob[cur, r, sl] = _round_nte(xb[cur, r, sl] - mv) + mv

        r0 = r_in_b + k * _CH
        hout[k] = [
            pltpu.async_copy(ob.at[cur], out_hbm.at[b, pl.ds(r0, _CH)], sem_out[cur]),
        ]
    for h in hout[_NCH - 2]:
        h.wait()
    for h in hout[_NCH - 1]:
        h.wait()


def _erfc_nonneg(a):
    t = 1.0 / (1.0 + _P * a)
    poly = t * (_A1 + t * (_A2 + t * (_A3 + t * (_A4 + t * _A5))))
    return poly * jnp.exp(-(a * a))


def _tc_body(x_ref, s_ref, m_ref, lik_ref):
    x = x_ref[...]
    s = s_ref[...]
    m = m_ref[...]
    v = jnp.abs(_round_nte(x - m))
    sb = jnp.maximum(s, SCALE_BOUND)
    inv = _INV_SQRT2 / sb
    a = (v + 0.5) * inv
    bz = (v - 0.5) * inv
    ea = _erfc_nonneg(a)
    eb_mag = _erfc_nonneg(jnp.abs(bz))
    eb = jnp.where(bz < 0.0, 2.0 - eb_mag, eb_mag)
    lik = 0.5 * (eb - ea)
    lik_ref[...] = jnp.maximum(lik, LIKELIHOOD_BOUND)


def kernel(x, scales, means):
    shape = x.shape
    r3 = (_B, _R, 128)
    x3 = x.reshape(r3)
    s3 = scales.reshape(r3)
    m3 = means.reshape(r3)

    bb, br = 2, 768
    spec = pl.BlockSpec((bb, br, 128), lambda i, j: (i, j, 0))
    lik = pl.pallas_call(
        _tc_body,
        grid=(_B // bb, _R // br),
        in_specs=[spec, spec, spec],
        out_specs=spec,
        out_shape=jax.ShapeDtypeStruct(r3, jnp.float32),
    )(x3, s3, m3)

    mesh = plsc.VectorSubcoreMesh(core_axis_name="c", subcore_axis_name="s")
    out = functools.partial(
        pl.kernel,
        mesh=mesh,
        compiler_params=pltpu.CompilerParams(use_tc_tiling_on_sc=True),
        out_type=jax.ShapeDtypeStruct(r3, jnp.float32),
        scratch_types=[
            pltpu.VMEM((2, _CH, 128), jnp.float32),
            pltpu.VMEM((2, _CH, 128), jnp.float32),
            pltpu.VMEM((2, _CH, 128), jnp.float32),
            pltpu.SemaphoreType.DMA,
            pltpu.SemaphoreType.DMA,
            pltpu.SemaphoreType.DMA,
            pltpu.SemaphoreType.DMA,
        ],
    )(_sc_body)(x3, m3)

    return out.reshape(shape), lik.reshape(shape)


# speedup vs baseline: 2.6375x; 2.6268x over previous
import jax
import jax.numpy as jnp
from jax.experimental import pallas as pl

SCALE_BOUND = 0.11
LIKELIHOOD_BOUND = 1e-09

_P = 0.3275911
_A1 = 0.254829592
_A2 = -0.284496736
_A3 = 1.421413741
_A4 = -1.453152027
_A5 = 1.061405429
_INV_SQRT2 = 0.7071067811865476


def _erfc_nonneg(a):
    t = 1.0 / (1.0 + _P * a)
    poly = t * (_A1 + t * (_A2 + t * (_A3 + t * (_A4 + t * _A5))))
    return poly * jnp.exp(-(a * a))


def _body(x_ref, s_ref, m_ref, out_ref, lik_ref):
    x = x_ref[...]
    s = s_ref[...]
    m = m_ref[...]
    q = jnp.round(x - m)
    out_ref[...] = q + m
    v = jnp.abs(q)
    sb = jnp.maximum(s, SCALE_BOUND)
    inv = _INV_SQRT2 / sb
    a = (v + 0.5) * inv
    b = (v - 0.5) * inv
    ea = _erfc_nonneg(a)
    eb_mag = _erfc_nonneg(jnp.abs(b))
    eb = jnp.where(b < 0.0, 2.0 - eb_mag, eb_mag)
    lik = 0.5 * (eb - ea)
    lik_ref[...] = jnp.maximum(lik, LIKELIHOOD_BOUND)


def kernel(x, scales, means):
    shape = x.shape
    b, c, h, w = shape
    r4 = (b, c, (h * w) // 128, 128)
    x4 = x.reshape(r4)
    s4 = scales.reshape(r4)
    m4 = means.reshape(r4)
    bb, bc = 2, 192
    grid = (b // bb, c // bc)
    spec = pl.BlockSpec((bb, bc, r4[2], 128), lambda i, j: (i, j, 0, 0))
    out, lik = pl.pallas_call(
        _body,
        grid=grid,
        in_specs=[spec, spec, spec],
        out_specs=[spec, spec],
        out_shape=[
            jax.ShapeDtypeStruct(r4, jnp.float32),
            jax.ShapeDtypeStruct(r4, jnp.float32),
        ],
    )(x4, s4, m4)
    return out.reshape(shape), lik.reshape(shape)
